# baseline (device time: 127807 ns/iter reference)
import jax
import jax.numpy as jnp
from jax import lax
from jax.experimental import pallas as pl
from jax.experimental.pallas import tpu as pltpu

B, S, D = 1, 1024, 2048
H, Dh, Dr = 16, 128, 32
DC = 128
SCALE = (Dh + Dr) ** -0.5
BF = jnp.bfloat16
F32 = jnp.float32
NWQ = 8
WQC = D // NWQ


def kernel(x, Wdkv, Wuk, Wuv, Wq, Wqr, Wkr, Wo):
    x, Wdkv, Wuk, Wuv, Wqr, Wkr = (
        a.astype(BF) for a in (x, Wdkv, Wuk, Wuv, Wqr, Wkr)
    )

    def body(
        x_ref, wdkv_ref, wuk_ref, wuv_ref, wq_hbm, wqr_ref, wkr_ref, wo_hbm,
        out_ref,
        c_send, c_recv, wk_recv, wv_recv, q_buf, k_buf, v_buf,
        wq_stage, wo_stage,
        send_sems, recv_sems, wq_sems, wo_sems,
    ):
        my_x = lax.axis_index("x")
        my_y = lax.axis_index("y")
        my_z = lax.axis_index("z")
        peer = (1 - my_x, my_y, my_z)

        def wq_dma(j):
            return pltpu.make_async_copy(
                wq_hbm.at[:, pl.ds(j * WQC, WQC)],
                wq_stage.at[j % 2],
                wq_sems.at[j % 2],
            )

        def wo_dma(h):
            return pltpu.make_async_copy(
                wo_hbm.at[pl.ds(h * Dh, Dh), :],
                wo_stage.at[h % 2],
                wo_sems.at[h % 2],
            )

        wq_dma(0).start()

        xb = x_ref[0]

        c_send[...] = jnp.dot(
            xb, wdkv_ref[...], preferred_element_type=F32
        ).astype(BF)

        barrier = pltpu.get_barrier_semaphore()
        pl.semaphore_signal(
            barrier, inc=1, device_id=peer, device_id_type=pl.DeviceIdType.MESH
        )
        pl.semaphore_wait(barrier, 1)

        rdmas = []
        pairs = [(c_send, c_recv), (wuk_ref, wk_recv), (wuv_ref, wv_recv)]
        for i, (src, dst) in enumerate(pairs):
            r = pltpu.make_async_remote_copy(
                src_ref=src,
                dst_ref=dst,
                send_sem=send_sems.at[i],
                recv_sem=recv_sems.at[i],
                device_id=peer,
                device_id_type=pl.DeviceIdType.MESH,
            )
            r.start()
            rdmas.append(r)

        for j in range(NWQ):
            wq_dma(j).wait()
            if j + 1 < NWQ:
                wq_dma(j + 1).start()
            q_buf[:, j * WQC:(j + 1) * WQC] = jnp.dot(
                xb, wq_stage[j % 2].astype(BF), preferred_element_type=F32
            ).astype(BF)

        Qr = jnp.dot(xb, wqr_ref[...], preferred_element_type=F32).astype(BF)
        Kr = jnp.dot(xb, wkr_ref[...], preferred_element_type=F32).astype(BF)
        c_loc = c_send[...]
        k_buf[...] = jnp.dot(c_loc, wuk_ref[...], preferred_element_type=F32).astype(BF)
        v_buf[...] = jnp.dot(c_loc, wuv_ref[...], preferred_element_type=F32).astype(BF)

        wo_dma(0).start()

        for r in rdmas:
            r.wait()

        k_buf[...] += jnp.dot(
            c_recv[...], wk_recv[...], preferred_element_type=F32
        ).astype(BF)
        v_buf[...] += jnp.dot(
            c_recv[...], wv_recv[...], preferred_element_type=F32
        ).astype(BF)

        contract = (((1,), (1,)), ((), ()))
        for h in range(H):
            wo_dma(h).wait()
            if h + 1 < H:
                wo_dma(h + 1).start()
            q = q_buf[:, h * Dh:(h + 1) * Dh]
            k = k_buf[:, h * Dh:(h + 1) * Dh]
            v = v_buf[:, h * Dh:(h + 1) * Dh]
            qr = Qr[:, h * Dr:(h + 1) * Dr]
            s = lax.dot_general(q, k, contract, preferred_element_type=F32)
            s = s + lax.dot_general(qr, Kr, contract, preferred_element_type=F32)
            p = jnp.exp(s * SCALE)
            denom = jnp.sum(p, axis=-1, keepdims=True)
            o_h = jnp.dot(p.astype(BF), v, preferred_element_type=F32)
            o_h = (o_h / denom).astype(BF)
            contrib = jnp.dot(
                o_h, wo_stage[h % 2].astype(BF), preferred_element_type=F32
            )
            if h == 0:
                out_ref[0] = contrib
            else:
                out_ref[0] += contrib

    return pl.pallas_call(
        body,
        out_shape=jax.ShapeDtypeStruct((B, S, D), F32),
        in_specs=[
            pl.BlockSpec(memory_space=pltpu.VMEM),
            pl.BlockSpec(memory_space=pltpu.VMEM),
            pl.BlockSpec(memory_space=pltpu.VMEM),
            pl.BlockSpec(memory_space=pltpu.VMEM),
            pl.BlockSpec(memory_space=pltpu.MemorySpace.HBM),
            pl.BlockSpec(memory_space=pltpu.VMEM),
            pl.BlockSpec(memory_space=pltpu.VMEM),
            pl.BlockSpec(memory_space=pltpu.MemorySpace.HBM),
        ],
        out_specs=pl.BlockSpec(memory_space=pltpu.VMEM),
        scratch_shapes=[
            pltpu.VMEM((S, DC), BF),
            pltpu.VMEM((S, DC), BF),
            pltpu.VMEM((DC, D), BF),
            pltpu.VMEM((DC, D), BF),
            pltpu.VMEM((S, D), BF),
            pltpu.VMEM((S, D), BF),
            pltpu.VMEM((S, D), BF),
            pltpu.VMEM((2, D, WQC), F32),
            pltpu.VMEM((2, Dh, D), F32),
            pltpu.SemaphoreType.DMA((3,)),
            pltpu.SemaphoreType.DMA((3,)),
            pltpu.SemaphoreType.DMA((2,)),
            pltpu.SemaphoreType.DMA((2,)),
        ],
        compiler_params=pltpu.CompilerParams(
            collective_id=0,
            vmem_limit_bytes=100 * 1024 * 1024,
        ),
    )(x, Wdkv, Wuk, Wuv, Wq, Wqr, Wkr, Wo)
